# Initial kernel scaffold; baseline (speedup 1.0000x reference)
import jax
import jax.numpy as jnp
from jax.experimental import pallas as pl


def kernel(node_rep, edge_rep, cycle_rep, edge_atoms, cycle5_atoms, cycle6_atoms, params):
    raise NotImplementedError("write your pallas kernel here")


# speedup vs baseline: 1.3399x; 1.3399x over previous
import functools

import jax
import jax.numpy as jnp
from jax.experimental import pallas as pl
from jax.experimental.pallas import tpu as pltpu

_N = 10000
_EPS = 1e-5
_BM = 1000


def _mm_stats_body(x_ref, w_ref, s_ref, t_ref, y_ref, st_ref=None, *,
                   pre_relu, with_stats):
    x = x_ref[...]
    if pre_relu:
        x = jnp.maximum(x * s_ref[...] + t_ref[...], 0.0)
    y = jnp.dot(x, w_ref[...], preferred_element_type=jnp.float32)
    y_ref[...] = y
    if with_stats:
        @pl.when(pl.program_id(0) == 0)
        def _init():
            st_ref[...] = jnp.zeros_like(st_ref)
        su = jnp.sum(y, axis=0, keepdims=True)
        sq = jnp.sum(y * y, axis=0, keepdims=True)
        st_ref[...] += jnp.concatenate([su, sq], axis=0)


@functools.partial(jax.jit, static_argnames=("pre_relu", "with_stats"))
def _mm(x, w, s=None, t=None, pre_relu=False, with_stats=True):
    m, k = x.shape
    f = w.shape[1]
    if s is None:
        s = jnp.ones((k,), jnp.float32)
        t = jnp.zeros((k,), jnp.float32)
    out_shape = [jax.ShapeDtypeStruct((m, f), jnp.float32)]
    out_specs = [pl.BlockSpec((_BM, f), lambda i: (i, 0))]
    if with_stats:
        out_shape.append(jax.ShapeDtypeStruct((2, f), jnp.float32))
        out_specs.append(pl.BlockSpec((2, f), lambda i: (0, 0)))
    res = pl.pallas_call(
        functools.partial(_mm_stats_body, pre_relu=pre_relu,
                          with_stats=with_stats),
        grid=(m // _BM,),
        in_specs=[
            pl.BlockSpec((_BM, k), lambda i: (i, 0)),
            pl.BlockSpec((k, f), lambda i: (0, 0)),
            pl.BlockSpec((1, k), lambda i: (0, 0)),
            pl.BlockSpec((1, k), lambda i: (0, 0)),
        ],
        out_specs=out_specs,
        out_shape=out_shape,
    )(x, w, s.reshape(1, k), t.reshape(1, k))
    if with_stats:
        return res[0], res[1]
    return res[0] if isinstance(res, (list, tuple)) else res


def _col_stats_body(x_ref, st_ref):
    x = x_ref[...]
    @pl.when(pl.program_id(0) == 0)
    def _init():
        st_ref[...] = jnp.zeros_like(st_ref)
    su = jnp.sum(x, axis=0, keepdims=True)
    sq = jnp.sum(x * x, axis=0, keepdims=True)
    st_ref[...] += jnp.concatenate([su, sq], axis=0)


@jax.jit
def _col_stats(x):
    m, f = x.shape
    return pl.pallas_call(
        _col_stats_body,
        grid=(m // _BM,),
        in_specs=[pl.BlockSpec((_BM, f), lambda i: (i, 0))],
        out_specs=pl.BlockSpec((2, f), lambda i: (0, 0)),
        out_shape=jax.ShapeDtypeStruct((2, f), jnp.float32),
    )(x)


def _affine_relu_body(x_ref, s_ref, t_ref, y_ref):
    y_ref[...] = jnp.maximum(x_ref[...] * s_ref[...] + t_ref[...], 0.0)


@jax.jit
def _affine_relu(x, s, t):
    m, f = x.shape
    return pl.pallas_call(
        _affine_relu_body,
        grid=(m // _BM,),
        in_specs=[
            pl.BlockSpec((_BM, f), lambda i: (i, 0)),
            pl.BlockSpec((1, f), lambda i: (0, 0)),
            pl.BlockSpec((1, f), lambda i: (0, 0)),
        ],
        out_specs=pl.BlockSpec((_BM, f), lambda i: (i, 0)),
        out_shape=jax.ShapeDtypeStruct((m, f), jnp.float32),
    )(x, s.reshape(1, f), t.reshape(1, f))


def _pool_bcast_body(x_ref, y_ref):
    x = x_ref[...]
    s = jnp.sum(x, axis=1, keepdims=True)
    y_ref[...] = jnp.broadcast_to(s, x.shape)


@functools.partial(jax.jit, static_argnames=("k",))
def _pool_bcast(x, k):
    m, c = x.shape
    i = m // k
    x3 = x.reshape(i, k, c)
    bi = 500
    y3 = pl.pallas_call(
        _pool_bcast_body,
        grid=(i // bi,),
        in_specs=[pl.BlockSpec((bi, k, c), lambda j: (j, 0, 0))],
        out_specs=pl.BlockSpec((bi, k, c), lambda j: (j, 0, 0)),
        out_shape=jax.ShapeDtypeStruct((i, k, c), jnp.float32),
    )(x3)
    return y3.reshape(m, c)


def _bn_st(st, m, g, b):
    mean = st[0] / m
    var = st[1] / m - mean * mean
    s = g * jax.lax.rsqrt(var + _EPS)
    return s, b - mean * s


def _seg(v, idx):
    return jax.ops.segment_sum(v, idx, num_segments=_N)


def _take(tab, idx):
    return jnp.take(tab, idx, axis=0)


def kernel(node_rep, edge_rep, cycle_rep, edge_atoms, cycle5_atoms,
           cycle6_atoms, params):
    p = params
    ea = edge_atoms
    c5a = cycle5_atoms
    c6a = cycle6_atoms
    re = ea.shape[0]
    r5 = c5a.shape[0]
    ca = jnp.concatenate([c5a, c6a])
    ea_sw = ea.reshape(-1, 2)[:, ::-1].reshape(-1)

    A = _seg(edge_rep, ea)
    deg = _seg(jnp.ones((re, 1), jnp.float32), ea)
    deg5 = _seg(jnp.ones((r5, 1), jnp.float32), c5a)
    deg6 = _seg(jnp.ones((c6a.shape[0], 1), jnp.float32), c6a)
    B = deg * node_rep

    gAe = _take(A, ea)
    gBe = _take(B, ea)
    gA5 = _take(A, c5a)
    gA6 = _take(A, c6a)

    P_A = _seg(gAe, ea_sw)
    acc2 = jnp.concatenate([deg * A, deg * A + P_A], axis=1)

    inst5_1 = _pool_bcast(gA5, 5)
    inst6_1 = _pool_bcast(gA6, 6)
    acc5 = jnp.concatenate([deg5 * A, _seg(inst5_1, c5a)], axis=1)
    acc6 = jnp.concatenate([deg6 * A, _seg(inst6_1, c6a)], axis=1)

    g2e = _take(acc2, ea)
    g5 = _take(acc5, c5a)
    g6 = _take(acc6, c6a)

    P2 = _seg(g2e, ea_sw)
    Pn = _seg(gBe, ea_sw)

    inst5b = _pool_bcast(g5, 5)
    inst6b = _pool_bcast(g6, 6)
    accC = jnp.concatenate([
        deg5 * acc5 + deg6 * acc6,
        _seg(inst5b, c5a) + _seg(inst6b, c6a),
        _seg(cycle_rep, ca),
    ], axis=1)

    dacc2 = deg * acc2
    dB = deg * B
    x_node = jnp.concatenate([node_rep, dacc2, dacc2 + P2, dB, dB + Pn],
                             axis=1)
    yn1, st = _mm(x_node, p['en_node_W1'])
    s1, t1 = _bn_st(st, _N, p['en_node_g1'], p['en_node_b1'])
    yn2, st = _mm(yn1, p['en_node_W2'], s1, t1, pre_relu=True)
    s2, t2 = _bn_st(st, _N, p['en_node_g2'], p['en_node_b2'])
    node_out = _affine_relu(yn2, s2, t2)

    W1ee = p['en_edge_W1']
    We = W1ee[:64]
    Wl2, Wi2 = W1ee[64:192], W1ee[192:320]
    Wln, Win = W1ee[320:384], W1ee[384:448]
    W1ec = p['ec_edge_W1']
    We2 = W1ec[:64]
    WCl, WCi = W1ec[64:384], W1ec[384:704]

    WUV = jnp.concatenate([
        jnp.concatenate([Wl2 + Wi2, Wln + Win], axis=0),
        jnp.concatenate([Wi2, Win], axis=0)], axis=1)
    uv = _mm(jnp.concatenate([acc2, B], axis=1), WUV, with_stats=False)
    ucvc = _mm(accC, jnp.concatenate([WCl + WCi, WCi], axis=1),
               with_stats=False)
    TP = jnp.concatenate([uv[:, :128], ucvc[:, :128]], axis=1)
    TQ = jnp.concatenate([uv[:, 128:], ucvc[:, 128:]], axis=1)

    base = _mm(edge_rep, jnp.concatenate([We, We2], axis=1),
               with_stats=False)
    y12 = base + _take(TP, ea) + _take(TQ, ea_sw)

    st = _col_stats(y12)
    sg = jnp.concatenate([p['en_edge_g1'], p['ec_edge_g1']])
    sb = jnp.concatenate([p['en_edge_b1'], p['ec_edge_b1']])
    s1, t1 = _bn_st(st, re, sg, sb)
    z64 = jnp.zeros((128, 64), jnp.float32)
    W2bd = jnp.concatenate([
        jnp.concatenate([p['en_edge_W2'], z64], axis=1),
        jnp.concatenate([z64, p['ec_edge_W2']], axis=1)], axis=0)
    y12b, st = _mm(y12, W2bd, s1, t1, pre_relu=True)
    sg = jnp.concatenate([p['en_edge_g2'], p['ec_edge_g2']])
    sb = jnp.concatenate([p['en_edge_b2'], p['ec_edge_b2']])
    s2, t2 = _bn_st(st, re, sg, sb)
    yc, st = _mm(y12b, p['conv_W'], s2, t2, pre_relu=True)
    sc, tc = _bn_st(st, re, p['conv_g'], p['conv_b'])
    edge_out = _affine_relu(yc, sc, tc)

    x_cyc = jnp.concatenate([
        jnp.concatenate([jnp.concatenate([g5, inst5b], axis=1),
                         jnp.concatenate([g6, inst6b], axis=1)], axis=0),
        cycle_rep], axis=1)
    rc = x_cyc.shape[0]
    yc1, st = _mm(x_cyc, p['ec_cycle_W1'])
    s1, t1 = _bn_st(st, rc, p['ec_cycle_g1'], p['ec_cycle_b1'])
    yc2, st = _mm(yc1, p['ec_cycle_W2'], s1, t1, pre_relu=True)
    s2, t2 = _bn_st(st, rc, p['ec_cycle_g2'], p['ec_cycle_b2'])
    cycle_out = _affine_relu(yc2, s2, t2)

    return (node_out, edge_out, cycle_out)


# speedup vs baseline: 2.1040x; 1.5702x over previous
import functools

import jax
import jax.numpy as jnp
from jax import lax
from jax.experimental import pallas as pl
from jax.experimental.pallas import tpu as pltpu
from jax.experimental.pallas import tpu_sc as plsc

_N = 10000
_EPS = 1e-5
_BM = 1000

_NC = 2
_NS = 16
_NW = _NC * _NS
_STRIPE = _N // _NS


def _split_work(m):
    for chunk in (128, 125, 120, 112, 100, 96, 80, 64):
        for nw in range(_NW, 0, -1):
            if m % (nw * chunk) == 0:
                nch = m // (nw * chunk)
                if nch <= 24:
                    return nw, chunk, nch
    raise ValueError(f"no work split for m={m}")


def _sc_mesh():
    return plsc.VectorSubcoreMesh(core_axis_name="c", subcore_axis_name="s")


def _sc_scatter_add(src, idx):
    m, c = src.shape
    nw, chunk, nch = _split_work(m)
    idx3 = idx.reshape(nw, nch, chunk)
    src3 = src.reshape(nw * nch, chunk, c)

    def body(src_hbm, idx_hbm, z_hbm, out_hbm, idx_v, buf, acc):
        cid = lax.axis_index("c")
        sid = lax.axis_index("s")
        wid = sid * _NC + cid

        @pl.when(sid == 0)
        def _zero():
            pltpu.sync_copy(z_hbm, acc)

        plsc.subcore_barrier()

        @pl.when(wid < nw)
        def _work():
            base = wid * nch
            pltpu.sync_copy(idx_hbm.at[wid], idx_v)
            for j in range(nch):
                pltpu.sync_copy(src_hbm.at[base + j], buf)
                pltpu.sync_copy(buf, acc.at[idx_v.at[j]], add=True)

        plsc.subcore_barrier()

        @pl.when(sid == 0)
        def _writeback():
            pltpu.sync_copy(acc, out_hbm.at[cid])

    out = pl.kernel(
        body,
        out_type=jax.ShapeDtypeStruct((_NC, _N, c), jnp.float32),
        mesh=_sc_mesh(),
        compiler_params=pltpu.CompilerParams(use_tc_tiling_on_sc=False),
        scratch_types=[
            pltpu.VMEM((nch, chunk), jnp.int32),
            pltpu.VMEM((chunk, c), jnp.float32),
            pltpu.VMEM_SHARED((_N, c), jnp.float32),
        ],
    )(src3, idx3, jnp.zeros((_N, c), jnp.float32))
    return out[0] + out[1]


def _sc_gather(tab, idx):
    n, c = tab.shape
    m = idx.shape[0]
    nw, chunk, nch = _split_work(m)
    idx3 = idx.reshape(nw, nch, chunk)

    def body(tab_hbm, idx_hbm, out_hbm, idx_v, buf, sem):
        cid = lax.axis_index("c")
        sid = lax.axis_index("s")
        wid = sid * _NC + cid

        @pl.when(wid < nw)
        def _work():
            base = wid * nch
            pltpu.sync_copy(idx_hbm.at[wid], idx_v)
            for j in range(nch):
                pltpu.async_copy(tab_hbm.at[idx_v.at[j]], buf, sem).wait()
                pltpu.sync_copy(buf, out_hbm.at[base + j])

    out = pl.kernel(
        body,
        out_type=jax.ShapeDtypeStruct((nw * nch, chunk, c), jnp.float32),
        mesh=_sc_mesh(),
        compiler_params=pltpu.CompilerParams(use_tc_tiling_on_sc=False),
        scratch_types=[
            pltpu.VMEM((nch, chunk), jnp.int32),
            pltpu.VMEM((chunk, c), jnp.float32),
            pltpu.SemaphoreType.DMA,
        ],
    )(tab, idx3)
    return out.reshape(m, c)


def _mm_stats_body(x_ref, w_ref, s_ref, t_ref, y_ref, st_ref=None, *,
                   pre_relu, with_stats):
    x = x_ref[...]
    if pre_relu:
        x = jnp.maximum(x * s_ref[...] + t_ref[...], 0.0)
    y = jnp.dot(x, w_ref[...], preferred_element_type=jnp.float32)
    y_ref[...] = y
    if with_stats:
        @pl.when(pl.program_id(0) == 0)
        def _init():
            st_ref[...] = jnp.zeros_like(st_ref)
        su = jnp.sum(y, axis=0, keepdims=True)
        sq = jnp.sum(y * y, axis=0, keepdims=True)
        st_ref[...] += jnp.concatenate([su, sq], axis=0)


@functools.partial(jax.jit, static_argnames=("pre_relu", "with_stats"))
def _mm(x, w, s=None, t=None, pre_relu=False, with_stats=True):
    m, k = x.shape
    f = w.shape[1]
    if s is None:
        s = jnp.ones((k,), jnp.float32)
        t = jnp.zeros((k,), jnp.float32)
    out_shape = [jax.ShapeDtypeStruct((m, f), jnp.float32)]
    out_specs = [pl.BlockSpec((_BM, f), lambda i: (i, 0))]
    if with_stats:
        out_shape.append(jax.ShapeDtypeStruct((2, f), jnp.float32))
        out_specs.append(pl.BlockSpec((2, f), lambda i: (0, 0)))
    res = pl.pallas_call(
        functools.partial(_mm_stats_body, pre_relu=pre_relu,
                          with_stats=with_stats),
        grid=(m // _BM,),
        in_specs=[
            pl.BlockSpec((_BM, k), lambda i: (i, 0)),
            pl.BlockSpec((k, f), lambda i: (0, 0)),
            pl.BlockSpec((1, k), lambda i: (0, 0)),
            pl.BlockSpec((1, k), lambda i: (0, 0)),
        ],
        out_specs=out_specs,
        out_shape=out_shape,
    )(x, w, s.reshape(1, k), t.reshape(1, k))
    if with_stats:
        return res[0], res[1]
    return res[0] if isinstance(res, (list, tuple)) else res


def _col_stats_body(x_ref, st_ref):
    x = x_ref[...]
    @pl.when(pl.program_id(0) == 0)
    def _init():
        st_ref[...] = jnp.zeros_like(st_ref)
    su = jnp.sum(x, axis=0, keepdims=True)
    sq = jnp.sum(x * x, axis=0, keepdims=True)
    st_ref[...] += jnp.concatenate([su, sq], axis=0)


@jax.jit
def _col_stats(x):
    m, f = x.shape
    return pl.pallas_call(
        _col_stats_body,
        grid=(m // _BM,),
        in_specs=[pl.BlockSpec((_BM, f), lambda i: (i, 0))],
        out_specs=pl.BlockSpec((2, f), lambda i: (0, 0)),
        out_shape=jax.ShapeDtypeStruct((2, f), jnp.float32),
    )(x)


def _affine_relu_body(x_ref, s_ref, t_ref, y_ref):
    y_ref[...] = jnp.maximum(x_ref[...] * s_ref[...] + t_ref[...], 0.0)


@jax.jit
def _affine_relu(x, s, t):
    m, f = x.shape
    return pl.pallas_call(
        _affine_relu_body,
        grid=(m // _BM,),
        in_specs=[
            pl.BlockSpec((_BM, f), lambda i: (i, 0)),
            pl.BlockSpec((1, f), lambda i: (0, 0)),
            pl.BlockSpec((1, f), lambda i: (0, 0)),
        ],
        out_specs=pl.BlockSpec((_BM, f), lambda i: (i, 0)),
        out_shape=jax.ShapeDtypeStruct((m, f), jnp.float32),
    )(x, s.reshape(1, f), t.reshape(1, f))


def _pool_bcast_body(x_ref, y_ref):
    x = x_ref[...]
    s = jnp.sum(x, axis=1, keepdims=True)
    y_ref[...] = jnp.broadcast_to(s, x.shape)


@functools.partial(jax.jit, static_argnames=("k",))
def _pool_bcast(x, k):
    m, c = x.shape
    i = m // k
    x3 = x.reshape(i, k, c)
    bi = 500
    y3 = pl.pallas_call(
        _pool_bcast_body,
        grid=(i // bi,),
        in_specs=[pl.BlockSpec((bi, k, c), lambda j: (j, 0, 0))],
        out_specs=pl.BlockSpec((bi, k, c), lambda j: (j, 0, 0)),
        out_shape=jax.ShapeDtypeStruct((i, k, c), jnp.float32),
    )(x3)
    return y3.reshape(m, c)


def _add3_stats_body(a_ref, b_ref, c_ref, y_ref, st_ref):
    y = a_ref[...] + b_ref[...] + c_ref[...]
    y_ref[...] = y
    @pl.when(pl.program_id(0) == 0)
    def _init():
        st_ref[...] = jnp.zeros_like(st_ref)
    su = jnp.sum(y, axis=0, keepdims=True)
    sq = jnp.sum(y * y, axis=0, keepdims=True)
    st_ref[...] += jnp.concatenate([su, sq], axis=0)


@jax.jit
def _add3_stats(a, b, c):
    m, f = a.shape
    return pl.pallas_call(
        _add3_stats_body,
        grid=(m // _BM,),
        in_specs=[pl.BlockSpec((_BM, f), lambda i: (i, 0))] * 3,
        out_specs=[
            pl.BlockSpec((_BM, f), lambda i: (i, 0)),
            pl.BlockSpec((2, f), lambda i: (0, 0)),
        ],
        out_shape=[
            jax.ShapeDtypeStruct((m, f), jnp.float32),
            jax.ShapeDtypeStruct((2, f), jnp.float32),
        ],
    )(a, b, c)


def _bn_st(st, m, g, b):
    mean = st[0] / m
    var = st[1] / m - mean * mean
    s = g * jax.lax.rsqrt(var + _EPS)
    return s, b - mean * s


def kernel(node_rep, edge_rep, cycle_rep, edge_atoms, cycle5_atoms,
           cycle6_atoms, params):
    p = params
    ea = edge_atoms
    c5a = cycle5_atoms
    c6a = cycle6_atoms
    re = ea.shape[0]
    r5 = c5a.shape[0]
    ca = jnp.concatenate([c5a, c6a])
    ea_sw = ea.reshape(-1, 2)[:, ::-1].reshape(-1)

    A_deg = _sc_scatter_add(
        jnp.concatenate([edge_rep, jnp.ones((re, 16), jnp.float32)], axis=1),
        ea)
    A = A_deg[:, :64]
    deg = A_deg[:, 64:65]
    B = deg * node_rep

    gAe = _sc_gather(A, ea)
    gAc = _sc_gather(A, ca)
    gA5 = gAc[:r5]
    gA6 = gAc[r5:]

    P_A = _sc_scatter_add(gAe, ea_sw)[:, :64]
    acc2 = jnp.concatenate([deg * A, deg * A + P_A], axis=1)

    inst5_1 = _pool_bcast(gA5, 5)
    inst6_1 = _pool_bcast(gA6, 6)
    s5 = _sc_scatter_add(
        jnp.concatenate([inst5_1, jnp.ones((r5, 16), jnp.float32)], axis=1),
        c5a)
    s6 = _sc_scatter_add(
        jnp.concatenate([inst6_1, jnp.ones((c6a.shape[0], 16), jnp.float32)],
                        axis=1), c6a)
    deg5 = s5[:, 64:65]
    deg6 = s6[:, 64:65]
    acc5 = jnp.concatenate([deg5 * A, s5[:, :64]], axis=1)
    acc6 = jnp.concatenate([deg6 * A, s6[:, :64]], axis=1)

    g2e = _sc_gather(acc2, ea)
    gBe = _sc_gather(B, ea)
    g5 = _sc_gather(acc5, c5a)
    g6 = _sc_gather(acc6, c6a)

    P2 = _sc_scatter_add(g2e, ea_sw)
    Pn = _sc_scatter_add(gBe, ea_sw)

    inst5b = _pool_bcast(g5, 5)
    inst6b = _pool_bcast(g6, 6)
    sc5 = _sc_scatter_add(inst5b, c5a)
    sc6 = _sc_scatter_add(inst6b, c6a)
    screp = _sc_scatter_add(cycle_rep, ca)
    accC = jnp.concatenate([
        deg5 * acc5 + deg6 * acc6,
        sc5 + sc6,
        screp,
    ], axis=1)

    dacc2 = deg * acc2
    dB = deg * B
    x_node = jnp.concatenate([node_rep, dacc2, dacc2 + P2, dB, dB + Pn],
                             axis=1)
    yn1, st = _mm(x_node, p['en_node_W1'])
    s1, t1 = _bn_st(st, _N, p['en_node_g1'], p['en_node_b1'])
    yn2, st = _mm(yn1, p['en_node_W2'], s1, t1, pre_relu=True)
    s2, t2 = _bn_st(st, _N, p['en_node_g2'], p['en_node_b2'])
    node_out = _affine_relu(yn2, s2, t2)

    W1ee = p['en_edge_W1']
    We = W1ee[:64]
    Wl2, Wi2 = W1ee[64:192], W1ee[192:320]
    Wln, Win = W1ee[320:384], W1ee[384:448]
    W1ec = p['ec_edge_W1']
    We2 = W1ec[:64]
    WCl, WCi = W1ec[64:384], W1ec[384:704]

    WUV = jnp.concatenate([
        jnp.concatenate([Wl2 + Wi2, Wln + Win], axis=0),
        jnp.concatenate([Wi2, Win], axis=0)], axis=1)
    uv = _mm(jnp.concatenate([acc2, B], axis=1), WUV, with_stats=False)
    ucvc = _mm(accC, jnp.concatenate([WCl + WCi, WCi], axis=1),
               with_stats=False)
    TP = jnp.concatenate([uv[:, :128], ucvc[:, :128]], axis=1)
    TQ = jnp.concatenate([uv[:, 128:], ucvc[:, 128:]], axis=1)

    base = _mm(edge_rep, jnp.concatenate([We, We2], axis=1),
               with_stats=False)
    y12, st = _add3_stats(base, _sc_gather(TP, ea), _sc_gather(TQ, ea_sw))
    sg = jnp.concatenate([p['en_edge_g1'], p['ec_edge_g1']])
    sb = jnp.concatenate([p['en_edge_b1'], p['ec_edge_b1']])
    s1, t1 = _bn_st(st, re, sg, sb)
    z64 = jnp.zeros((128, 64), jnp.float32)
    W2bd = jnp.concatenate([
        jnp.concatenate([p['en_edge_W2'], z64], axis=1),
        jnp.concatenate([z64, p['ec_edge_W2']], axis=1)], axis=0)
    y12b, st = _mm(y12, W2bd, s1, t1, pre_relu=True)
    sg = jnp.concatenate([p['en_edge_g2'], p['ec_edge_g2']])
    sb = jnp.concatenate([p['en_edge_b2'], p['ec_edge_b2']])
    s2, t2 = _bn_st(st, re, sg, sb)
    yc, st = _mm(y12b, p['conv_W'], s2, t2, pre_relu=True)
    sc, tc = _bn_st(st, re, p['conv_g'], p['conv_b'])
    edge_out = _affine_relu(yc, sc, tc)

    x_cyc = jnp.concatenate([
        jnp.concatenate([jnp.concatenate([g5, inst5b], axis=1),
                         jnp.concatenate([g6, inst6b], axis=1)], axis=0),
        cycle_rep], axis=1)
    rc = x_cyc.shape[0]
    yc1, st = _mm(x_cyc, p['ec_cycle_W1'])
    s1, t1 = _bn_st(st, rc, p['ec_cycle_g1'], p['ec_cycle_b1'])
    yc2, st = _mm(yc1, p['ec_cycle_W2'], s1, t1, pre_relu=True)
    s2, t2 = _bn_st(st, rc, p['ec_cycle_g2'], p['ec_cycle_b2'])
    cycle_out = _affine_relu(yc2, s2, t2)

    return (node_out, edge_out, cycle_out)
